# Initial kernel scaffold; baseline (speedup 1.0000x reference)
import jax
import jax.numpy as jnp
from jax.experimental import pallas as pl


def kernel(field0, field1, field2):
    raise NotImplementedError("write your pallas kernel here")


# speedup vs baseline: 1.2391x; 1.2391x over previous
import functools

import jax
import jax.numpy as jnp
from jax import lax
from jax.experimental import pallas as pl
from jax.experimental.pallas import tpu as pltpu
from jax.experimental.pallas import tpu_sc as plsc

_B = 64
_J = 19
_W = 4096
_NW = 32


def _fi_of(j):
    return jnp.where(
        j < 4, j ^ 2,
        jnp.where(
            j == 4, j,
            jnp.where(
                j < 7, j ^ 3,
                jnp.where(
                    j == 7, j,
                    jnp.where(
                        j < 12, j ^ 1,
                        jnp.where(j == 12, j, ((j - 13) ^ 1) + 13))))))


def _is_rev(j):
    return (j == 4) | (j == 7) | (j == 12)


def _flip_image(inbuf, outbuf, sign):
    sgn = jnp.full((16,), 1.0, jnp.float32) * sign

    def chunk(v, _):
        x = inbuf[pl.ds((v ^ 3) * 16, 16)]
        outbuf[pl.ds(v * 16, 16)] = lax.rev(x, (0,)) * sgn
        return _

    lax.fori_loop(0, _W // 16, chunk, None)


def _sc_hflip(f0, f1, f2):
    mesh = plsc.VectorSubcoreMesh(core_axis_name="c", subcore_axis_name="s")
    img = jnp.float32

    @functools.partial(
        pl.kernel,
        out_type=(
            jax.ShapeDtypeStruct((_B * _J * _W,), img),
            jax.ShapeDtypeStruct((_B * _J * 2 * _W,), img),
            jax.ShapeDtypeStruct((_B * _J * 2 * _W,), img),
        ),
        mesh=mesh,
        scratch_types=(
            pltpu.VMEM((_W,), img),
            pltpu.VMEM((_W,), img),
        ),
    )
    def k(f0h, f1h, f2h, o0h, o1h, o2h, inbuf, outbuf):
        wid = lax.axis_index("c") * 16 + lax.axis_index("s")

        def copy_flip(src_hbm, src_off, dst_hbm, dst_off, sign):
            pltpu.sync_copy(src_hbm.at[pl.ds(src_off * _W, _W)], inbuf)
            _flip_image(inbuf, outbuf, sign)
            pltpu.sync_copy(outbuf, dst_hbm.at[pl.ds(dst_off * _W, _W)])

        def per_b(b_i, _):
            b = wid * 2 + b_i

            def per_j0(j, _):
                copy_flip(f0h, b * _J + _fi_of(j), o0h, b * _J + j, 1.0)
                return _

            lax.fori_loop(0, _J, per_j0, None)

            def per_jc(t, _):
                j = t >> 1
                c = t & 1
                fj = _fi_of(j)
                sign = jnp.where(c == 0, -1.0, 1.0).astype(jnp.float32)
                src = (b * _J + fj) * 2 + c
                dst = (b * _J + j) * 2 + c
                rev = _is_rev(j)

                @pl.when(jnp.logical_not(rev))
                def _fwd():
                    copy_flip(f1h, src, o1h, dst, sign)
                    copy_flip(f2h, src, o2h, dst, sign)

                @pl.when(rev)
                def _swp():
                    copy_flip(f2h, src, o1h, dst, sign)
                    copy_flip(f1h, src, o2h, dst, sign)

                return _

            lax.fori_loop(0, _J * 2, per_jc, None)
            return _

        lax.fori_loop(0, 2, per_b, None)

    return k(f0, f1, f2)


def kernel(field0, field1, field2):
    o0, o1, o2 = _sc_hflip(
        field0.reshape(-1), field1.reshape(-1), field2.reshape(-1))
    return (
        o0.reshape(field0.shape),
        o1.reshape(field1.shape),
        o2.reshape(field2.shape),
    )
